# initial kernel scaffold (unmeasured)
import jax
import jax.numpy as jnp
from jax import lax
from jax.experimental import pallas as pl
from jax.experimental.pallas import tpu as pltpu


def kernel(
    x,
):
    def body(*refs):
        pass

    out_shape = jax.ShapeDtypeStruct(..., jnp.float32)
    return pl.pallas_call(body, out_shape=out_shape)(...)


# baseline (device time: 56985 ns/iter reference)
import jax
import jax.numpy as jnp
from jax import lax
from jax.experimental import pallas as pl
from jax.experimental.pallas import tpu as pltpu

M = 2048
HALF_M = 1024
N_OUT = 512


def kernel(x):
    def body(x_ref, out_ref, send_buf, xrecv, yrecv, send_sems, recv_sems):
        my_x = lax.axis_index("x")
        my_y = lax.axis_index("y")
        other_x = 1 - my_x
        other_y = 1 - my_y

        barrier_sem = pltpu.get_barrier_semaphore()
        pl.semaphore_signal(
            barrier_sem, inc=1,
            device_id=(other_x, my_y), device_id_type=pl.DeviceIdType.MESH,
        )
        pl.semaphore_signal(
            barrier_sem, inc=1,
            device_id=(my_x, other_y), device_id_type=pl.DeviceIdType.MESH,
        )
        pl.semaphore_wait(barrier_sem, 2)

        row0 = my_y * HALF_M

        @pl.when(my_x == 0)
        def _():
            send_buf[:, :] = x_ref[0, pl.ds(row0, HALF_M), N_OUT:2 * N_OUT]

        @pl.when(my_x == 1)
        def _():
            send_buf[:, :] = x_ref[0, pl.ds(row0, HALF_M), 0:N_OUT]

        rdma_x = pltpu.make_async_remote_copy(
            src_ref=send_buf,
            dst_ref=xrecv,
            send_sem=send_sems.at[0],
            recv_sem=recv_sems.at[0],
            device_id=(other_x, my_y),
            device_id_type=pl.DeviceIdType.MESH,
        )
        rdma_x.start()
        rdma_x.wait()

        rdma_y = pltpu.make_async_remote_copy(
            src_ref=xrecv,
            dst_ref=yrecv,
            send_sem=send_sems.at[1],
            recv_sem=recv_sems.at[1],
            device_id=(my_x, other_y),
            device_id_type=pl.DeviceIdType.MESH,
        )
        rdma_y.start()
        rdma_y.wait()

        @pl.when(my_x == 0)
        def _():
            out_ref[pl.ds(row0, HALF_M), :] = (
                x_ref[0, pl.ds(row0, HALF_M), 0:N_OUT] + xrecv[:, :]
            )
            out_ref[pl.ds(other_y * HALF_M, HALF_M), :] = (
                x_ref[0, pl.ds(other_y * HALF_M, HALF_M), 0:N_OUT] + yrecv[:, :]
            )

        @pl.when(my_x == 1)
        def _():
            out_ref[pl.ds(row0, HALF_M), :] = (
                x_ref[0, pl.ds(row0, HALF_M), N_OUT:2 * N_OUT] + xrecv[:, :]
            )
            out_ref[pl.ds(other_y * HALF_M, HALF_M), :] = (
                x_ref[0, pl.ds(other_y * HALF_M, HALF_M), N_OUT:2 * N_OUT]
                + yrecv[:, :]
            )

    return pl.pallas_call(
        body,
        out_shape=jax.ShapeDtypeStruct((M, N_OUT), jnp.float32),
        in_specs=[pl.BlockSpec(memory_space=pltpu.VMEM)],
        out_specs=pl.BlockSpec(memory_space=pltpu.VMEM),
        scratch_shapes=[
            pltpu.VMEM((HALF_M, N_OUT), jnp.float32),
            pltpu.VMEM((HALF_M, N_OUT), jnp.float32),
            pltpu.VMEM((HALF_M, N_OUT), jnp.float32),
            pltpu.SemaphoreType.DMA((2,)),
            pltpu.SemaphoreType.DMA((2,)),
        ],
        compiler_params=pltpu.CompilerParams(collective_id=0),
    )(x)


# device time: 38200 ns/iter; 1.4918x vs baseline; 1.4918x over previous
import functools

import jax
import jax.numpy as jnp
from jax import lax
from jax.experimental import pallas as pl
from jax.experimental.pallas import tpu as pltpu

M = 2048
HALF_M = 1024
N_OUT = 512
S = 8
CH = HALF_M // S


def kernel(x):
    def body(x_ref, out_ref, xrecv, yrecv, sx_sems, rx_sems, sy_sems, ry_sems):
        my_x = lax.axis_index("x")
        my_y = lax.axis_index("y")
        other_x = 1 - my_x
        other_y = 1 - my_y

        barrier_sem = pltpu.get_barrier_semaphore()
        pl.semaphore_signal(
            barrier_sem, inc=1,
            device_id=(other_x, my_y), device_id_type=pl.DeviceIdType.MESH,
        )
        pl.semaphore_signal(
            barrier_sem, inc=1,
            device_id=(my_x, other_y), device_id_type=pl.DeviceIdType.MESH,
        )
        pl.semaphore_wait(barrier_sem, 2)

        row0 = my_y * HALF_M
        orow0 = other_y * HALF_M

        def pipeline(mx):
            ox = 1 - mx
            send_cols = slice(ox * N_OUT, (ox + 1) * N_OUT)
            keep_cols = slice(mx * N_OUT, (mx + 1) * N_OUT)

            x_rd = []
            for k in range(S):
                r = pltpu.make_async_remote_copy(
                    src_ref=x_ref.at[0, pl.ds(row0 + k * CH, CH), send_cols],
                    dst_ref=xrecv.at[pl.ds(k * CH, CH)],
                    send_sem=sx_sems.at[k],
                    recv_sem=rx_sems.at[k],
                    device_id=(ox, my_y),
                    device_id_type=pl.DeviceIdType.MESH,
                )
                r.start()
                x_rd.append(r)

            y_rd = []
            for k in range(S):
                x_rd[k].wait_recv()
                ry = pltpu.make_async_remote_copy(
                    src_ref=xrecv.at[pl.ds(k * CH, CH)],
                    dst_ref=yrecv.at[pl.ds(k * CH, CH)],
                    send_sem=sy_sems.at[k],
                    recv_sem=ry_sems.at[k],
                    device_id=(mx, other_y),
                    device_id_type=pl.DeviceIdType.MESH,
                )
                ry.start()
                y_rd.append(ry)
                out_ref[pl.ds(row0 + k * CH, CH), :] = (
                    x_ref[0, pl.ds(row0 + k * CH, CH), keep_cols]
                    + xrecv[pl.ds(k * CH, CH), :]
                )

            for k in range(S):
                y_rd[k].wait_recv()
                out_ref[pl.ds(orow0 + k * CH, CH), :] = (
                    x_ref[0, pl.ds(orow0 + k * CH, CH), keep_cols]
                    + yrecv[pl.ds(k * CH, CH), :]
                )

            for k in range(S):
                x_rd[k].wait_send()
                y_rd[k].wait_send()

        @pl.when(my_x == 0)
        def _():
            pipeline(0)

        @pl.when(my_x == 1)
        def _():
            pipeline(1)

        @functools.partial(
            pl.run_scoped, second_barrier=pltpu.SemaphoreType.REGULAR
        )
        def _(second_barrier):
            pl.semaphore_signal(
                second_barrier, inc=1,
                device_id=(other_x, my_y), device_id_type=pl.DeviceIdType.MESH,
            )
            pl.semaphore_signal(
                second_barrier, inc=1,
                device_id=(my_x, other_y), device_id_type=pl.DeviceIdType.MESH,
            )
            pl.semaphore_wait(second_barrier, 2)

    return pl.pallas_call(
        body,
        out_shape=jax.ShapeDtypeStruct((M, N_OUT), jnp.float32),
        in_specs=[pl.BlockSpec(memory_space=pltpu.VMEM)],
        out_specs=pl.BlockSpec(memory_space=pltpu.VMEM),
        scratch_shapes=[
            pltpu.VMEM((HALF_M, N_OUT), jnp.float32),
            pltpu.VMEM((HALF_M, N_OUT), jnp.float32),
            pltpu.SemaphoreType.DMA((S,)),
            pltpu.SemaphoreType.DMA((S,)),
            pltpu.SemaphoreType.DMA((S,)),
            pltpu.SemaphoreType.DMA((S,)),
        ],
        compiler_params=pltpu.CompilerParams(collective_id=0),
    )(x)


# device time: 35878 ns/iter; 1.5883x vs baseline; 1.0647x over previous
import jax
import jax.numpy as jnp
from jax import lax
from jax.experimental import pallas as pl
from jax.experimental.pallas import tpu as pltpu

M = 2048
HALF_M = 1024
N_OUT = 512
S = 16
CH = HALF_M // S


def kernel(x):
    def body(x_ref, out_ref, sbuf, xrecv, yrecv,
             sx_sems, rx_sems, sy_sems, ry_sems):
        my_x = lax.axis_index("x")
        my_y = lax.axis_index("y")
        other_x = 1 - my_x
        other_y = 1 - my_y

        barrier_sem = pltpu.get_barrier_semaphore()
        pl.semaphore_signal(
            barrier_sem, inc=1,
            device_id=(other_x, my_y), device_id_type=pl.DeviceIdType.MESH,
        )
        pl.semaphore_signal(
            barrier_sem, inc=1,
            device_id=(my_x, other_y), device_id_type=pl.DeviceIdType.MESH,
        )
        pl.semaphore_wait(barrier_sem, 2)

        row0 = my_y * HALF_M
        orow0 = other_y * HALF_M

        def pipeline(mx):
            ox = 1 - mx
            send_cols = slice(ox * N_OUT, (ox + 1) * N_OUT)
            keep_cols = slice(mx * N_OUT, (mx + 1) * N_OUT)

            x_rd = []
            for k in range(S):
                ck = pl.ds(k * CH, CH)
                sbuf[ck, :] = x_ref[0, pl.ds(row0 + k * CH, CH), send_cols]
                r = pltpu.make_async_remote_copy(
                    src_ref=sbuf.at[ck],
                    dst_ref=xrecv.at[ck],
                    send_sem=sx_sems.at[k],
                    recv_sem=rx_sems.at[k],
                    device_id=(ox, my_y),
                    device_id_type=pl.DeviceIdType.MESH,
                )
                r.start()
                x_rd.append(r)

            y_rd = []
            for k in range(S):
                ck = pl.ds(k * CH, CH)
                x_rd[k].wait_recv()
                ry = pltpu.make_async_remote_copy(
                    src_ref=xrecv.at[ck],
                    dst_ref=yrecv.at[ck],
                    send_sem=sy_sems.at[k],
                    recv_sem=ry_sems.at[k],
                    device_id=(mx, other_y),
                    device_id_type=pl.DeviceIdType.MESH,
                )
                ry.start()
                y_rd.append(ry)
                out_ref[pl.ds(row0 + k * CH, CH), :] = (
                    x_ref[0, pl.ds(row0 + k * CH, CH), keep_cols]
                    + xrecv[ck, :]
                )

            for k in range(S):
                ck = pl.ds(k * CH, CH)
                y_rd[k].wait_recv()
                out_ref[pl.ds(orow0 + k * CH, CH), :] = (
                    x_ref[0, pl.ds(orow0 + k * CH, CH), keep_cols]
                    + yrecv[ck, :]
                )

            for k in range(S):
                x_rd[k].wait_send()
                y_rd[k].wait_send()

        @pl.when(my_x == 0)
        def _():
            pipeline(0)

        @pl.when(my_x == 1)
        def _():
            pipeline(1)

    return pl.pallas_call(
        body,
        out_shape=jax.ShapeDtypeStruct((M, N_OUT), jnp.float32),
        in_specs=[pl.BlockSpec(memory_space=pltpu.VMEM)],
        out_specs=pl.BlockSpec(memory_space=pltpu.VMEM),
        scratch_shapes=[
            pltpu.VMEM((HALF_M, N_OUT), jnp.float32),
            pltpu.VMEM((HALF_M, N_OUT), jnp.float32),
            pltpu.VMEM((HALF_M, N_OUT), jnp.float32),
            pltpu.SemaphoreType.DMA((S,)),
            pltpu.SemaphoreType.DMA((S,)),
            pltpu.SemaphoreType.DMA((S,)),
            pltpu.SemaphoreType.DMA((S,)),
        ],
        compiler_params=pltpu.CompilerParams(collective_id=0),
    )(x)
